# Initial kernel scaffold; baseline (speedup 1.0000x reference)
import jax
import jax.numpy as jnp
from jax.experimental import pallas as pl


def kernel(inputs, targets):
    raise NotImplementedError("write your pallas kernel here")


# speedup vs baseline: 1.6910x; 1.6910x over previous
import jax
import jax.numpy as jnp
from jax.experimental import pallas as pl


def _pass_a(x_ref, t_ref, nll_ref, hist_ref):
    x = x_ref[...]
    r, c = x.shape
    t = t_ref[0, 0, :]
    m = jnp.max(x, axis=1, keepdims=True)
    s = jnp.sum(jnp.exp(x - m), axis=1)
    lse = jnp.log(s) + m[:, 0]
    cols = jax.lax.broadcasted_iota(jnp.int32, (r, c), 1)
    maskf = (cols == t[:, None]).astype(jnp.float32)
    tl = jnp.sum(x * maskf, axis=1)
    nll_ref[0, 0, :] = lse - tl
    hpart = jnp.sum(maskf, axis=0)[None, :]

    @pl.when(pl.program_id(0) == 0)
    def _():
        hist_ref[...] = hpart

    @pl.when(pl.program_id(0) > 0)
    def _():
        hist_ref[...] += hpart


def _pass_b(n_total, n_steps, hist_ref, t_ref, nll_ref, out_ref):
    h = hist_ref[0, :]
    freq = h * (1.0 / n_total)
    a = 1.0 / (freq + 1e-5)
    alpha = a / jnp.sum(a)
    t = t_ref[0, 0, :]
    nll = nll_ref[0, 0, :]
    r = t.shape[0]
    c = alpha.shape[0]
    cols = jax.lax.broadcasted_iota(jnp.int32, (r, c), 1)
    maskf = (cols == t[:, None]).astype(jnp.float32)
    ag = jnp.sum(maskf * alpha[None, :], axis=1)
    ce = ag * nll
    pt = jnp.exp(-ce)
    om = 1.0 - pt
    ps = jnp.broadcast_to(jnp.sum(om * om * ce), (1, 1))
    i = pl.program_id(0)

    @pl.when(i == 0)
    def _():
        out_ref[...] = ps

    @pl.when(i > 0)
    def _():
        out_ref[...] += ps

    @pl.when(i == n_steps - 1)
    def _():
        out_ref[...] *= 1.0 / n_total


def kernel(inputs, targets):
    n, c = inputs.shape
    ra = 512
    ga = n // ra
    t3a = targets.astype(jnp.int32).reshape(ga, 1, ra)

    nll, hist = pl.pallas_call(
        _pass_a,
        grid=(ga,),
        in_specs=[
            pl.BlockSpec((ra, c), lambda i: (i, 0)),
            pl.BlockSpec((1, 1, ra), lambda i: (i, 0, 0)),
        ],
        out_specs=[
            pl.BlockSpec((1, 1, ra), lambda i: (i, 0, 0)),
            pl.BlockSpec((1, c), lambda i: (0, 0)),
        ],
        out_shape=[
            jax.ShapeDtypeStruct((ga, 1, ra), jnp.float32),
            jax.ShapeDtypeStruct((1, c), jnp.float32),
        ],
    )(inputs, t3a)

    rb = 1024
    gb = n // rb
    t3b = targets.astype(jnp.int32).reshape(gb, 1, rb)
    nll3b = nll.reshape(gb, 1, rb)

    loss = pl.pallas_call(
        lambda *refs: _pass_b(n, gb, *refs),
        grid=(gb,),
        in_specs=[
            pl.BlockSpec((1, c), lambda i: (0, 0)),
            pl.BlockSpec((1, 1, rb), lambda i: (i, 0, 0)),
            pl.BlockSpec((1, 1, rb), lambda i: (i, 0, 0)),
        ],
        out_specs=pl.BlockSpec((1, 1), lambda i: (0, 0)),
        out_shape=jax.ShapeDtypeStruct((1, 1), jnp.float32),
    )(hist, t3b, nll3b)

    return loss[0, 0]


# speedup vs baseline: 1.8609x; 1.1005x over previous
import jax
import jax.numpy as jnp
from jax.experimental import pallas as pl


def _pass_a(x_ref, t_ref, nll_ref, hist_ref):
    x = x_ref[...]
    r, c = x.shape
    t = t_ref[0, 0, :]
    m = jnp.max(x, axis=1, keepdims=True)
    s = jnp.sum(jnp.exp(x - m), axis=1)
    lse = jnp.log(s) + m[:, 0]
    cols = jax.lax.broadcasted_iota(jnp.int32, (r, c), 1)
    maskf = (cols == t[:, None]).astype(jnp.float32)
    tl = jnp.sum(x * maskf, axis=1)
    nll_ref[0, 0, :] = lse - tl
    hpart = jnp.sum(maskf, axis=0)[None, :]

    @pl.when(pl.program_id(0) == 0)
    def _():
        hist_ref[...] = hpart

    @pl.when(pl.program_id(0) > 0)
    def _():
        hist_ref[...] += hpart


def _pass_b(n_total, n_steps, hist_ref, t_ref, nll_ref, out_ref):
    h = hist_ref[0, :]
    freq = h * (1.0 / n_total)
    a = 1.0 / (freq + 1e-5)
    alpha = a / jnp.sum(a)
    t = t_ref[0, 0, :]
    nll = nll_ref[0, 0, :]
    r = t.shape[0]
    c = alpha.shape[0]
    cols = jax.lax.broadcasted_iota(jnp.int32, (r, c), 1)
    maskf = (cols == t[:, None]).astype(jnp.float32)
    ag = jnp.sum(maskf * alpha[None, :], axis=1)
    ce = ag * nll
    pt = jnp.exp(-ce)
    om = 1.0 - pt
    ps = jnp.broadcast_to(jnp.sum(om * om * ce), (1, 1))
    i = pl.program_id(0)

    @pl.when(i == 0)
    def _():
        out_ref[...] = ps

    @pl.when(i > 0)
    def _():
        out_ref[...] += ps

    @pl.when(i == n_steps - 1)
    def _():
        out_ref[...] *= 1.0 / n_total


def kernel(inputs, targets):
    n, c = inputs.shape
    ra = 1024
    ga = n // ra
    t3a = targets.astype(jnp.int32).reshape(ga, 1, ra)

    nll, hist = pl.pallas_call(
        _pass_a,
        grid=(ga,),
        in_specs=[
            pl.BlockSpec((ra, c), lambda i: (i, 0)),
            pl.BlockSpec((1, 1, ra), lambda i: (i, 0, 0)),
        ],
        out_specs=[
            pl.BlockSpec((1, 1, ra), lambda i: (i, 0, 0)),
            pl.BlockSpec((1, c), lambda i: (0, 0)),
        ],
        out_shape=[
            jax.ShapeDtypeStruct((ga, 1, ra), jnp.float32),
            jax.ShapeDtypeStruct((1, c), jnp.float32),
        ],
    )(inputs, t3a)

    rb = 2048
    gb = n // rb
    t3b = targets.astype(jnp.int32).reshape(gb, 1, rb)
    nll3b = nll.reshape(gb, 1, rb)

    loss = pl.pallas_call(
        lambda *refs: _pass_b(n, gb, *refs),
        grid=(gb,),
        in_specs=[
            pl.BlockSpec((1, c), lambda i: (0, 0)),
            pl.BlockSpec((1, 1, rb), lambda i: (i, 0, 0)),
            pl.BlockSpec((1, 1, rb), lambda i: (i, 0, 0)),
        ],
        out_specs=pl.BlockSpec((1, 1), lambda i: (0, 0)),
        out_shape=jax.ShapeDtypeStruct((1, 1), jnp.float32),
    )(hist, t3b, nll3b)

    return loss[0, 0]


# speedup vs baseline: 1.8746x; 1.0074x over previous
import jax
import jax.numpy as jnp
from jax.experimental import pallas as pl


def _pass_a(x_ref, t_ref, nll_ref, hist_ref):
    x = x_ref[...]
    r, c = x.shape
    t = t_ref[0, 0, :]
    m = jnp.max(x, axis=1, keepdims=True)
    s = jnp.sum(jnp.exp(x - m), axis=1)
    lse = jnp.log(s) + m[:, 0]
    cols = jax.lax.broadcasted_iota(jnp.int32, (r, c), 1)
    maskf = (cols == t[:, None]).astype(jnp.float32)
    tl = jnp.sum(x * maskf, axis=1)
    nll_ref[0, 0, :] = lse - tl
    hpart = jnp.sum(maskf, axis=0)[None, :]

    @pl.when(pl.program_id(0) == 0)
    def _():
        hist_ref[...] = hpart

    @pl.when(pl.program_id(0) > 0)
    def _():
        hist_ref[...] += hpart


def _pass_b(n_total, n_steps, hist_ref, t_ref, nll_ref, out_ref):
    h = hist_ref[0, :]
    freq = h * (1.0 / n_total)
    a = 1.0 / (freq + 1e-5)
    alpha = a / jnp.sum(a)
    t = t_ref[0, 0, :]
    nll = nll_ref[0, 0, :]
    r = t.shape[0]
    c = alpha.shape[0]
    cols = jax.lax.broadcasted_iota(jnp.int32, (r, c), 1)
    maskf = (cols == t[:, None]).astype(jnp.float32)
    ag = jnp.sum(maskf * alpha[None, :], axis=1)
    ce = ag * nll
    pt = jnp.exp(-ce)
    om = 1.0 - pt
    ps = jnp.broadcast_to(jnp.sum(om * om * ce), (1, 1))
    i = pl.program_id(0)

    @pl.when(i == 0)
    def _():
        out_ref[...] = ps

    @pl.when(i > 0)
    def _():
        out_ref[...] += ps

    @pl.when(i == n_steps - 1)
    def _():
        out_ref[...] *= 1.0 / n_total


def kernel(inputs, targets):
    n, c = inputs.shape
    ra = 2048
    ga = n // ra
    t3a = targets.astype(jnp.int32).reshape(ga, 1, ra)

    nll, hist = pl.pallas_call(
        _pass_a,
        grid=(ga,),
        in_specs=[
            pl.BlockSpec((ra, c), lambda i: (i, 0)),
            pl.BlockSpec((1, 1, ra), lambda i: (i, 0, 0)),
        ],
        out_specs=[
            pl.BlockSpec((1, 1, ra), lambda i: (i, 0, 0)),
            pl.BlockSpec((1, c), lambda i: (0, 0)),
        ],
        out_shape=[
            jax.ShapeDtypeStruct((ga, 1, ra), jnp.float32),
            jax.ShapeDtypeStruct((1, c), jnp.float32),
        ],
    )(inputs, t3a)

    rb = 4096
    gb = n // rb
    t3b = targets.astype(jnp.int32).reshape(gb, 1, rb)
    nll3b = nll.reshape(gb, 1, rb)

    loss = pl.pallas_call(
        lambda *refs: _pass_b(n, gb, *refs),
        grid=(gb,),
        in_specs=[
            pl.BlockSpec((1, c), lambda i: (0, 0)),
            pl.BlockSpec((1, 1, rb), lambda i: (i, 0, 0)),
            pl.BlockSpec((1, 1, rb), lambda i: (i, 0, 0)),
        ],
        out_specs=pl.BlockSpec((1, 1), lambda i: (0, 0)),
        out_shape=jax.ShapeDtypeStruct((1, 1), jnp.float32),
    )(hist, t3b, nll3b)

    return loss[0, 0]
